# baseline (speedup 1.0000x reference)
import functools

import jax
import jax.numpy as jnp
from jax import lax
from jax.experimental import pallas as pl
from jax.experimental.pallas import tpu as pltpu
from jax.experimental.pallas import tpu_sc as plsc

N = 10000
E = 320000
F = 128
HID = 16
T = 12
TOUT = 4

NC = 2
NS = 16
CH = 128
NB = 4
NP = 10112
RPT = NP // NS
DISCARD = NP - 1

PH = 3
WH = PH * 2 * HID
CPT_A = 160
EPT_A = CPT_A * CH
EPC_A = EPT_A * NS

WB = 2 * HID
EH = E // 2
CPT_B = 80
EPT_B = CPT_B * CH
EPC_B = EPT_B * NS

WD = 16

BN = 1000


def _sc_edge_scatter(width, halves, cpt):
    mesh = plsc.VectorSubcoreMesh(
        core_axis_name="c", subcore_axis_name="s",
        num_cores=NC, num_subcores=NS)

    @functools.partial(
        pl.kernel,
        out_type=jax.ShapeDtypeStruct((NC, halves, NP, width), jnp.float32),
        mesh=mesh,
        scratch_types=[
            pltpu.VMEM((CH,), jnp.int32),
            pltpu.VMEM((CH,), jnp.int32),
            pltpu.VMEM((CH, width), jnp.float32),
            pltpu.VMEM_SHARED((NP, width), jnp.float32),
            pltpu.SemaphoreType.DMA,
        ],
        compiler_params=pltpu.CompilerParams(use_tc_tiling_on_sc=False),
    )
    def k(g, src, dst, zeros, out, sidx, didx, rows, acc, sem):
        c = lax.axis_index("c")
        s = lax.axis_index("s")
        r0 = s * RPT
        base = s * (cpt * CH)
        for h in range(halves):
            pltpu.sync_copy(zeros, acc.at[pl.ds(r0, RPT)])
            plsc.subcore_barrier()

            def chunk(i, carry):
                off = base + i * CH
                pltpu.sync_copy(src.at[c, h, pl.ds(off, CH)], sidx)
                pltpu.sync_copy(dst.at[c, pl.ds(off, CH)], didx)
                pltpu.async_copy(g.at[sidx], rows, sem).wait()
                pltpu.sync_copy(rows, acc.at[didx], add=True)
                return carry

            lax.fori_loop(0, cpt, chunk, 0)
            plsc.subcore_barrier()
            pltpu.sync_copy(acc.at[pl.ds(r0, RPT)],
                            out.at[c, h, pl.ds(r0, RPT)])

    return k


def _sc_degree():
    mesh = plsc.VectorSubcoreMesh(
        core_axis_name="c", subcore_axis_name="s",
        num_cores=NC, num_subcores=NS)

    @functools.partial(
        pl.kernel,
        out_type=jax.ShapeDtypeStruct((NC, NP, WD), jnp.float32),
        mesh=mesh,
        scratch_types=[
            pltpu.VMEM((CPT_B, CH), jnp.int32),
            pltpu.VMEM((CH, WD), jnp.float32),
            pltpu.VMEM_SHARED((NP, WD), jnp.float32),
        ],
        compiler_params=pltpu.CompilerParams(use_tc_tiling_on_sc=False),
    )
    def k(dst, zeros, ones, out, dst_v, ones_v, acc):
        c = lax.axis_index("c")
        s = lax.axis_index("s")
        r0 = s * RPT
        pltpu.sync_copy(dst.at[c, s], dst_v)
        pltpu.sync_copy(ones, ones_v)
        pltpu.sync_copy(zeros, acc.at[pl.ds(r0, RPT)])
        plsc.subcore_barrier()

        def chunk(i, carry):
            pltpu.sync_copy(ones_v, acc.at[dst_v.at[i]], add=True)
            return carry

        lax.fori_loop(0, CPT_B, chunk, 0)
        plsc.subcore_barrier()
        pltpu.sync_copy(acc.at[pl.ds(r0, RPT)], out.at[c, pl.ds(r0, RPT)])

    return k


def _full_spec(arr):
    shape = arr.shape
    return pl.BlockSpec(shape, lambda i: (0,) * len(shape))


def _acat(wz, lzw, wh, lhw):
    az = jnp.dot(wz, lzw[:HID], preferred_element_type=jnp.float32)
    ah = jnp.dot(wh, lhw[:HID], preferred_element_type=jnp.float32)
    return jnp.concatenate([az, ah], axis=1)


def _softmax12(att):
    m = jnp.max(att)
    e = jnp.exp(att - m)
    return e / jnp.sum(e)


def _gate(sg, bz, lzw, lzb, bh, lhw, lhb):
    cz = jnp.dot(bz, lzw[:HID], preferred_element_type=jnp.float32) + lzb
    ch = jnp.dot(bh, lhw[:HID], preferred_element_type=jnp.float32) + lhb
    z = jax.nn.sigmoid(sg[:, :HID] + cz)
    ht = jnp.tanh(sg[:, HID:] + ch)
    return (1.0 - z) * ht


def _tc_einsum_scale(wz, lzw, wh, lhw, xt, degp):
    def body(wz_r, lzw_r, wh_r, lhw_r, xt_r, degp_r, ga_r, dinv_r):
        acat = _acat(wz_r[...], lzw_r[...], wh_r[...], lhw_r[...])
        parts = [
            jnp.dot(xt_r[q], acat, preferred_element_type=jnp.float32)
            for q in range(T)
        ]
        deg = degp_r[0] + degp_r[1] + 1.0
        dv = lax.rsqrt(deg)
        dinv_r[...] = dv
        for g2 in range(4):
            ga_r[g2] = jnp.concatenate(parts[PH * g2:PH * (g2 + 1)],
                                       axis=1) * dv

    return pl.pallas_call(
        body,
        grid=(N // BN,),
        in_specs=[
            _full_spec(wz), _full_spec(lzw), _full_spec(wh), _full_spec(lhw),
            pl.BlockSpec((T, BN, F), lambda i: (0, i, 0)),
            pl.BlockSpec((NC, BN, 1), lambda i: (0, i, 0)),
        ],
        out_specs=[
            pl.BlockSpec((4, BN, WH), lambda i: (0, i, 0)),
            pl.BlockSpec((BN, 1), lambda i: (i, 0)),
        ],
        out_shape=[
            jax.ShapeDtypeStruct((4, N, WH), jnp.float32),
            jax.ShapeDtypeStruct((N, 1), jnp.float32),
        ],
    )(wz, lzw, wh, lhw, xt, degp)


def _tc_combine(scat, ga, dinv, att, bz, lzb, bh, lhb, lzw, lhw, wz, wh,
                outw, outb):
    def body(scat_r, ga_r, dinv_r, att_r, bz_r, lzb_r, bh_r, lhb_r,
             lzw_r, lhw_r, wz_r, wh_r, outw_r, outb_r,
             h_r, g12_r, pout_r):
        probs = _softmax12(att_r[...])
        dv = dinv_r[...]
        s = []
        for q in range(T):
            g2, pp = q // PH, q % PH
            sg = (scat_r[g2][:, pp * WB:(pp + 1) * WB]
                  + ga_r[g2][:, pp * WB:(pp + 1) * WB]) * dv
            s.append(_gate(sg, bz_r[...], lzw_r[...], lzb_r[...],
                           bh_r[...], lhw_r[...], lhb_r[...]))
        ps = []
        for t in range(TOUT):
            acc = jnp.zeros_like(s[0])
            for q in range(t, T):
                acc = acc + probs[0:1, q - t:q - t + 1] * s[q]
            ps.append(acc)
        h = jnp.dot(jax.nn.relu(ps[0]), outw_r[...],
                    preferred_element_type=jnp.float32) + outb_r[...]
        h_r[...] = h
        acat = _acat(wz_r[...], lzw_r[...], wh_r[...], lhw_r[...])
        g12_r[...] = jnp.dot(h, acat, preferred_element_type=jnp.float32) * dv
        for t in range(1, TOUT):
            pout_r[t - 1] = ps[t]

    return pl.pallas_call(
        body,
        grid=(N // BN,),
        in_specs=[
            pl.BlockSpec((2 * NC, BN, WH), lambda i: (0, i, 0)),
            pl.BlockSpec((4, BN, WH), lambda i: (0, i, 0)),
            pl.BlockSpec((BN, 1), lambda i: (i, 0)),
            _full_spec(att), _full_spec(bz), _full_spec(lzb),
            _full_spec(bh), _full_spec(lhb), _full_spec(lzw),
            _full_spec(lhw), _full_spec(wz), _full_spec(wh),
            _full_spec(outw), _full_spec(outb),
        ],
        out_specs=[
            pl.BlockSpec((BN, F), lambda i: (i, 0)),
            pl.BlockSpec((BN, WB), lambda i: (i, 0)),
            pl.BlockSpec((TOUT - 1, BN, HID), lambda i: (0, i, 0)),
        ],
        out_shape=[
            jax.ShapeDtypeStruct((N, F), jnp.float32),
            jax.ShapeDtypeStruct((N, WB), jnp.float32),
            jax.ShapeDtypeStruct((TOUT - 1, N, HID), jnp.float32),
        ],
    )(scat, ga, dinv, att, bz, lzb, bh, lhb, lzw, lhw, wz, wh, outw, outb)


def _tc_step(scatb, gt, dinv, pin, att, bz, lzb, bh, lhb, lzw, lhw, wz, wh,
             outw, outb, n_p, emit_g):
    def body(*refs):
        (scatb_r, gt_r, dinv_r, pin_r, att_r, bz_r, lzb_r, bh_r, lhb_r,
         lzw_r, lhw_r, wz_r, wh_r, outw_r, outb_r) = refs[:15]
        outs = refs[15:]
        h_r = outs[0]
        probs = _softmax12(att_r[...])
        dv = dinv_r[...]
        sg = (scatb_r[0, 0] + scatb_r[1, 0] + gt_r[...]) * dv
        s_new = _gate(sg, bz_r[...], lzw_r[...], lzb_r[...],
                      bh_r[...], lhw_r[...], lhb_r[...])
        h = jnp.dot(jax.nn.relu(pin_r[0] + probs[0:1, 11:12] * s_new),
                    outw_r[...], preferred_element_type=jnp.float32)
        h = h + outb_r[...]
        h_r[...] = h
        o = 1
        if n_p > 1:
            pout_r = outs[o]
            o += 1
            for j in range(n_p - 1):
                pout_r[j] = pin_r[j + 1] + probs[0:1, 10 - j:11 - j] * s_new
        if emit_g:
            acat = _acat(wz_r[...], lzw_r[...], wh_r[...], lhw_r[...])
            outs[o][...] = jnp.dot(
                h, acat, preferred_element_type=jnp.float32) * dv

    out_specs = [pl.BlockSpec((BN, F), lambda i: (i, 0))]
    out_shape = [jax.ShapeDtypeStruct((N, F), jnp.float32)]
    if n_p > 1:
        out_specs.append(pl.BlockSpec((n_p - 1, BN, HID), lambda i: (0, i, 0)))
        out_shape.append(jax.ShapeDtypeStruct((n_p - 1, N, HID), jnp.float32))
    if emit_g:
        out_specs.append(pl.BlockSpec((BN, WB), lambda i: (i, 0)))
        out_shape.append(jax.ShapeDtypeStruct((N, WB), jnp.float32))

    return pl.pallas_call(
        body,
        grid=(N // BN,),
        in_specs=[
            pl.BlockSpec((NC, 1, BN, WB), lambda i: (0, 0, i, 0)),
            pl.BlockSpec((BN, WB), lambda i: (i, 0)),
            pl.BlockSpec((BN, 1), lambda i: (i, 0)),
            pl.BlockSpec((n_p, BN, HID), lambda i: (0, i, 0)),
            _full_spec(att), _full_spec(bz), _full_spec(lzb),
            _full_spec(bh), _full_spec(lhb), _full_spec(lzw),
            _full_spec(lhw), _full_spec(wz), _full_spec(wh),
            _full_spec(outw), _full_spec(outb),
        ],
        out_specs=out_specs,
        out_shape=out_shape,
    )(scatb, gt, dinv, pin, att, bz, lzb, bh, lhb, lzw, lhw, wz, wh,
      outw, outb)


def _pad_flat(v, total, fill):
    pad = jnp.full((total - v.shape[0],), fill, jnp.int32)
    return jnp.concatenate([v, pad])


def _pad_reshape(v, total, fill):
    return _pad_flat(v, total, fill).reshape(NS, -1, CH)


def kernel(x, edge_index, Wz, bz, lzW, lzb, Wr, br, lrW, lrb, Wh, bh, lhW,
           lhb, att, outW, outb):
    x = x.astype(jnp.float32)
    src = edge_index[0].astype(jnp.int32)
    dst = edge_index[1].astype(jnp.int32)

    att2 = att.reshape(1, T)
    bz2 = bz.reshape(1, HID)
    lzb2 = lzb.reshape(1, HID)
    bh2 = bh.reshape(1, HID)
    lhb2 = lhb.reshape(1, HID)
    outb2 = outb.reshape(1, F)

    src_a = jnp.stack([
        jnp.stack([_pad_flat(src + (2 * c + h) * N, EPC_A, 0)
                   for h in range(2)])
        for c in range(NC)
    ])
    dst_a1 = _pad_flat(dst, EPC_A, DISCARD)
    dst_a = jnp.stack([dst_a1, dst_a1])

    src_b = jnp.stack([
        _pad_flat(src[:EH], EPC_B, 0)[None],
        _pad_flat(src[EH:], EPC_B, 0)[None],
    ])
    dst_b = jnp.stack([
        _pad_flat(dst[:EH], EPC_B, DISCARD),
        _pad_flat(dst[EH:], EPC_B, DISCARD),
    ])
    dst_d = jnp.stack([
        _pad_reshape(dst[:EH], EPC_B, DISCARD),
        _pad_reshape(dst[EH:], EPC_B, DISCARD),
    ])

    zeros_a = jnp.zeros((RPT, WH), jnp.float32)
    zeros_b = jnp.zeros((RPT, WB), jnp.float32)
    zeros_d = jnp.zeros((RPT, WD), jnp.float32)
    ones_d = jnp.ones((CH, WD), jnp.float32)

    degp_full = _sc_degree()(dst_d, zeros_d, ones_d)
    degp = degp_full[:, :N, 0:1]

    xt = jnp.transpose(x, (2, 0, 1))
    ga, dinv = _tc_einsum_scale(Wz, lzW, Wh, lhW, xt, degp)
    gaf = ga.reshape(4 * N, WH)
    scat12 = _sc_edge_scatter(WH, 2, CPT_A)(gaf, src_a, dst_a, zeros_a)
    scat12 = scat12.reshape(2 * NC, NP, WH)[:, :N]

    h1, g12, pp3 = _tc_combine(scat12, ga, dinv, att2, bz2, lzb2, bh2, lhb2,
                               lzW, lhW, Wz, Wh, outW, outb2)

    sc_b = _sc_edge_scatter(WB, 1, CPT_B)

    sb12 = sc_b(g12, src_b, dst_b, zeros_b)[:, :, :N]
    h2, pp2, g13 = _tc_step(sb12, g12, dinv, pp3, att2, bz2, lzb2, bh2, lhb2,
                            lzW, lhW, Wz, Wh, outW, outb2, 3, True)

    sb13 = sc_b(g13, src_b, dst_b, zeros_b)[:, :, :N]
    h3, pp1, g14 = _tc_step(sb13, g13, dinv, pp2, att2, bz2, lzb2, bh2, lhb2,
                            lzW, lhW, Wz, Wh, outW, outb2, 2, True)

    sb14 = sc_b(g14, src_b, dst_b, zeros_b)[:, :, :N]
    (h4,) = _tc_step(sb14, g14, dinv, pp1, att2, bz2, lzb2, bh2, lhb2,
                     lzW, lhW, Wz, Wh, outW, outb2, 1, False)

    return jnp.stack([h1, h2, h3, h4], axis=2)


# speedup vs baseline: 1.4519x; 1.4519x over previous
import functools

import jax
import jax.numpy as jnp
from jax import lax
from jax.experimental import pallas as pl
from jax.experimental.pallas import tpu as pltpu
from jax.experimental.pallas import tpu_sc as plsc

N = 10000
E = 320000
F = 128
HID = 16
T = 12
TOUT = 4

NC = 2
NS = 16
CH = 128
NB = 4
NP = 10112
RPT = NP // NS
DISCARD = NP - 1

PH = 3
WH = PH * 2 * HID
CPT_A = 160
EPT_A = CPT_A * CH
EPC_A = EPT_A * NS

WB = 2 * HID
EH = E // 2
CPT_B = 80
EPT_B = CPT_B * CH
EPC_B = EPT_B * NS

WD = 16

BN = 1000


def _sc_edge_scatter(width, halves, cpt):
    mesh = plsc.VectorSubcoreMesh(
        core_axis_name="c", subcore_axis_name="s",
        num_cores=NC, num_subcores=NS)

    @functools.partial(
        pl.kernel,
        out_type=jax.ShapeDtypeStruct((NC, halves, NP, width), jnp.float32),
        mesh=mesh,
        scratch_types=[
            pltpu.VMEM((CH,), jnp.int32),
            pltpu.VMEM((CH,), jnp.int32),
            pltpu.VMEM((CH, width), jnp.float32),
            pltpu.VMEM_SHARED((NP, width), jnp.float32),
            pltpu.SemaphoreType.DMA,
        ],
        compiler_params=pltpu.CompilerParams(use_tc_tiling_on_sc=False),
    )
    def k(g, src, dst, zeros, out, sidx, didx, rows, acc, sem):
        c = lax.axis_index("c")
        s = lax.axis_index("s")
        r0 = s * RPT
        base = s * (cpt * CH)
        for h in range(halves):
            pltpu.sync_copy(zeros, acc.at[pl.ds(r0, RPT)])
            plsc.subcore_barrier()

            def chunk(i, carry):
                off = base + i * CH
                pltpu.sync_copy(src.at[c, h, pl.ds(off, CH)], sidx)
                pltpu.sync_copy(dst.at[c, pl.ds(off, CH)], didx)
                pltpu.async_copy(g.at[sidx], rows, sem).wait()
                pltpu.sync_copy(rows, acc.at[didx], add=True)
                return carry

            lax.fori_loop(0, cpt, chunk, 0)
            plsc.subcore_barrier()
            pltpu.sync_copy(acc.at[pl.ds(r0, RPT)],
                            out.at[c, h, pl.ds(r0, RPT)])

    return k


def _sc_degree():
    mesh = plsc.VectorSubcoreMesh(
        core_axis_name="c", subcore_axis_name="s",
        num_cores=NC, num_subcores=NS)

    @functools.partial(
        pl.kernel,
        out_type=jax.ShapeDtypeStruct((NC, NP, WD), jnp.float32),
        mesh=mesh,
        scratch_types=[
            pltpu.VMEM((CPT_B, CH), jnp.int32),
            pltpu.VMEM((CH, WD), jnp.float32),
            pltpu.VMEM_SHARED((NP, WD), jnp.float32),
        ],
        compiler_params=pltpu.CompilerParams(use_tc_tiling_on_sc=False),
    )
    def k(dst, zeros, ones, out, dst_v, ones_v, acc):
        c = lax.axis_index("c")
        s = lax.axis_index("s")
        r0 = s * RPT
        pltpu.sync_copy(dst.at[c, s], dst_v)
        pltpu.sync_copy(ones, ones_v)
        pltpu.sync_copy(zeros, acc.at[pl.ds(r0, RPT)])
        plsc.subcore_barrier()

        def chunk(i, carry):
            pltpu.sync_copy(ones_v, acc.at[dst_v.at[i]], add=True)
            return carry

        lax.fori_loop(0, CPT_B, chunk, 0)
        plsc.subcore_barrier()
        pltpu.sync_copy(acc.at[pl.ds(r0, RPT)], out.at[c, pl.ds(r0, RPT)])

    return k


def _full_spec(arr):
    shape = arr.shape
    return pl.BlockSpec(shape, lambda i: (0,) * len(shape))


def _acat(wz, lzw, wh, lhw):
    az = jnp.dot(wz, lzw[:HID], preferred_element_type=jnp.float32)
    ah = jnp.dot(wh, lhw[:HID], preferred_element_type=jnp.float32)
    return jnp.concatenate([az, ah], axis=1)


def _softmax12(att):
    m = jnp.max(att)
    e = jnp.exp(att - m)
    return e / jnp.sum(e)


def _gate(sg, bz, lzw, lzb, bh, lhw, lhb):
    cz = jnp.dot(bz, lzw[:HID], preferred_element_type=jnp.float32) + lzb
    ch = jnp.dot(bh, lhw[:HID], preferred_element_type=jnp.float32) + lhb
    z = jax.nn.sigmoid(sg[:, :HID] + cz)
    ht = jnp.tanh(sg[:, HID:] + ch)
    return (1.0 - z) * ht


def _tc_einsum_scale(wz, lzw, wh, lhw, xt, degp):
    def body(wz_r, lzw_r, wh_r, lhw_r, xt_r, degp_r, ga_r, dinv_r):
        acat = _acat(wz_r[...], lzw_r[...], wh_r[...], lhw_r[...])
        parts = [
            jnp.dot(xt_r[q], acat, preferred_element_type=jnp.float32)
            for q in range(T)
        ]
        deg = degp_r[0] + degp_r[1] + 1.0
        dv = lax.rsqrt(deg)
        dinv_r[...] = dv
        for g2 in range(4):
            ga_r[g2] = jnp.concatenate(parts[PH * g2:PH * (g2 + 1)],
                                       axis=1) * dv

    return pl.pallas_call(
        body,
        grid=(N // BN,),
        in_specs=[
            _full_spec(wz), _full_spec(lzw), _full_spec(wh), _full_spec(lhw),
            pl.BlockSpec((T, BN, F), lambda i: (0, i, 0)),
            pl.BlockSpec((NC, BN, 1), lambda i: (0, i, 0)),
        ],
        out_specs=[
            pl.BlockSpec((4, BN, WH), lambda i: (0, i, 0)),
            pl.BlockSpec((BN, 1), lambda i: (i, 0)),
        ],
        out_shape=[
            jax.ShapeDtypeStruct((4, N, WH), jnp.float32),
            jax.ShapeDtypeStruct((N, 1), jnp.float32),
        ],
    )(wz, lzw, wh, lhw, xt, degp)


def _tc_combine(scat, ga, dinv, att, bz, lzb, bh, lhb, lzw, lhw, wz, wh,
                outw, outb):
    def body(scat_r, ga_r, dinv_r, att_r, bz_r, lzb_r, bh_r, lhb_r,
             lzw_r, lhw_r, wz_r, wh_r, outw_r, outb_r,
             h_r, g12_r, pout_r):
        probs = _softmax12(att_r[...])
        dv = dinv_r[...]
        s = []
        for q in range(T):
            g2, pp = q // PH, q % PH
            sg = (scat_r[g2][:, pp * WB:(pp + 1) * WB]
                  + ga_r[g2][:, pp * WB:(pp + 1) * WB]) * dv
            s.append(_gate(sg, bz_r[...], lzw_r[...], lzb_r[...],
                           bh_r[...], lhw_r[...], lhb_r[...]))
        ps = []
        for t in range(TOUT):
            acc = jnp.zeros_like(s[0])
            for q in range(t, T):
                acc = acc + probs[0:1, q - t:q - t + 1] * s[q]
            ps.append(acc)
        h = jnp.dot(jax.nn.relu(ps[0]), outw_r[...],
                    preferred_element_type=jnp.float32) + outb_r[...]
        h_r[...] = h
        acat = _acat(wz_r[...], lzw_r[...], wh_r[...], lhw_r[...])
        g12_r[...] = jnp.dot(h, acat, preferred_element_type=jnp.float32) * dv
        for t in range(1, TOUT):
            pout_r[t - 1] = ps[t]

    return pl.pallas_call(
        body,
        grid=(N // BN,),
        in_specs=[
            pl.BlockSpec((2 * NC, BN, WH), lambda i: (0, i, 0)),
            pl.BlockSpec((4, BN, WH), lambda i: (0, i, 0)),
            pl.BlockSpec((BN, 1), lambda i: (i, 0)),
            _full_spec(att), _full_spec(bz), _full_spec(lzb),
            _full_spec(bh), _full_spec(lhb), _full_spec(lzw),
            _full_spec(lhw), _full_spec(wz), _full_spec(wh),
            _full_spec(outw), _full_spec(outb),
        ],
        out_specs=[
            pl.BlockSpec((BN, F), lambda i: (i, 0)),
            pl.BlockSpec((BN, WB), lambda i: (i, 0)),
            pl.BlockSpec((TOUT - 1, BN, HID), lambda i: (0, i, 0)),
        ],
        out_shape=[
            jax.ShapeDtypeStruct((N, F), jnp.float32),
            jax.ShapeDtypeStruct((N, WB), jnp.float32),
            jax.ShapeDtypeStruct((TOUT - 1, N, HID), jnp.float32),
        ],
    )(scat, ga, dinv, att, bz, lzb, bh, lhb, lzw, lhw, wz, wh, outw, outb)


def _tc_step(scatb, gt, dinv, pin, att, bz, lzb, bh, lhb, lzw, lhw, wz, wh,
             outw, outb, n_p, emit_g):
    def body(*refs):
        (scatb_r, gt_r, dinv_r, pin_r, att_r, bz_r, lzb_r, bh_r, lhb_r,
         lzw_r, lhw_r, wz_r, wh_r, outw_r, outb_r) = refs[:15]
        outs = refs[15:]
        h_r = outs[0]
        probs = _softmax12(att_r[...])
        dv = dinv_r[...]
        sg = (scatb_r[0, 0] + scatb_r[1, 0] + gt_r[...]) * dv
        s_new = _gate(sg, bz_r[...], lzw_r[...], lzb_r[...],
                      bh_r[...], lhw_r[...], lhb_r[...])
        h = jnp.dot(jax.nn.relu(pin_r[0] + probs[0:1, 11:12] * s_new),
                    outw_r[...], preferred_element_type=jnp.float32)
        h = h + outb_r[...]
        h_r[...] = h
        o = 1
        if n_p > 1:
            pout_r = outs[o]
            o += 1
            for j in range(n_p - 1):
                pout_r[j] = pin_r[j + 1] + probs[0:1, 10 - j:11 - j] * s_new
        if emit_g:
            acat = _acat(wz_r[...], lzw_r[...], wh_r[...], lhw_r[...])
            outs[o][...] = jnp.dot(
                h, acat, preferred_element_type=jnp.float32) * dv

    out_specs = [pl.BlockSpec((BN, F), lambda i: (i, 0))]
    out_shape = [jax.ShapeDtypeStruct((N, F), jnp.float32)]
    if n_p > 1:
        out_specs.append(pl.BlockSpec((n_p - 1, BN, HID), lambda i: (0, i, 0)))
        out_shape.append(jax.ShapeDtypeStruct((n_p - 1, N, HID), jnp.float32))
    if emit_g:
        out_specs.append(pl.BlockSpec((BN, WB), lambda i: (i, 0)))
        out_shape.append(jax.ShapeDtypeStruct((N, WB), jnp.float32))

    return pl.pallas_call(
        body,
        grid=(N // BN,),
        in_specs=[
            pl.BlockSpec((NC, 1, BN, WB), lambda i: (0, 0, i, 0)),
            pl.BlockSpec((BN, WB), lambda i: (i, 0)),
            pl.BlockSpec((BN, 1), lambda i: (i, 0)),
            pl.BlockSpec((n_p, BN, HID), lambda i: (0, i, 0)),
            _full_spec(att), _full_spec(bz), _full_spec(lzb),
            _full_spec(bh), _full_spec(lhb), _full_spec(lzw),
            _full_spec(lhw), _full_spec(wz), _full_spec(wh),
            _full_spec(outw), _full_spec(outb),
        ],
        out_specs=out_specs,
        out_shape=out_shape,
    )(scatb, gt, dinv, pin, att, bz, lzb, bh, lhb, lzw, lhw, wz, wh,
      outw, outb)


def _pad_flat(v, total, base, mod):
    n_pad = total - v.shape[0]
    pad = base + (jnp.arange(n_pad, dtype=jnp.int32) % mod)
    return jnp.concatenate([v, pad])


def _pad_reshape(v, total, base, mod):
    return _pad_flat(v, total, base, mod).reshape(NS, -1, CH)


def kernel(x, edge_index, Wz, bz, lzW, lzb, Wr, br, lrW, lrb, Wh, bh, lhW,
           lhb, att, outW, outb):
    x = x.astype(jnp.float32)
    src = edge_index[0].astype(jnp.int32)
    dst = edge_index[1].astype(jnp.int32)

    att2 = att.reshape(1, T)
    bz2 = bz.reshape(1, HID)
    lzb2 = lzb.reshape(1, HID)
    bh2 = bh.reshape(1, HID)
    lhb2 = lhb.reshape(1, HID)
    outb2 = outb.reshape(1, F)

    n_disc = NP - N
    src_a = jnp.stack([
        jnp.stack([_pad_flat(src + (2 * c + h) * N, EPC_A,
                             (2 * c + h) * N, N)
                   for h in range(2)])
        for c in range(NC)
    ])
    dst_a1 = _pad_flat(dst, EPC_A, N, n_disc)
    dst_a = jnp.stack([dst_a1, dst_a1])

    src_b = jnp.stack([
        _pad_flat(src[:EH], EPC_B, 0, N)[None],
        _pad_flat(src[EH:], EPC_B, 0, N)[None],
    ])
    dst_b = jnp.stack([
        _pad_flat(dst[:EH], EPC_B, N, n_disc),
        _pad_flat(dst[EH:], EPC_B, N, n_disc),
    ])
    dst_d = jnp.stack([
        _pad_reshape(dst[:EH], EPC_B, N, n_disc),
        _pad_reshape(dst[EH:], EPC_B, N, n_disc),
    ])

    zeros_a = jnp.zeros((RPT, WH), jnp.float32)
    zeros_b = jnp.zeros((RPT, WB), jnp.float32)
    zeros_d = jnp.zeros((RPT, WD), jnp.float32)
    ones_d = jnp.ones((CH, WD), jnp.float32)

    degp_full = _sc_degree()(dst_d, zeros_d, ones_d)
    degp = degp_full[:, :N, 0:1]

    xt = jnp.transpose(x, (2, 0, 1))
    ga, dinv = _tc_einsum_scale(Wz, lzW, Wh, lhW, xt, degp)
    gaf = ga.reshape(4 * N, WH)
    scat12 = _sc_edge_scatter(WH, 2, CPT_A)(gaf, src_a, dst_a, zeros_a)
    scat12 = scat12.reshape(2 * NC, NP, WH)[:, :N]

    h1, g12, pp3 = _tc_combine(scat12, ga, dinv, att2, bz2, lzb2, bh2, lhb2,
                               lzW, lhW, Wz, Wh, outW, outb2)

    sc_b = _sc_edge_scatter(WB, 1, CPT_B)

    sb12 = sc_b(g12, src_b, dst_b, zeros_b)[:, :, :N]
    h2, pp2, g13 = _tc_step(sb12, g12, dinv, pp3, att2, bz2, lzb2, bh2, lhb2,
                            lzW, lhW, Wz, Wh, outW, outb2, 3, True)

    sb13 = sc_b(g13, src_b, dst_b, zeros_b)[:, :, :N]
    h3, pp1, g14 = _tc_step(sb13, g13, dinv, pp2, att2, bz2, lzb2, bh2, lhb2,
                            lzW, lhW, Wz, Wh, outW, outb2, 2, True)

    sb14 = sc_b(g14, src_b, dst_b, zeros_b)[:, :, :N]
    (h4,) = _tc_step(sb14, g14, dinv, pp1, att2, bz2, lzb2, bh2, lhb2,
                     lzW, lhW, Wz, Wh, outW, outb2, 1, False)

    return jnp.stack([h1, h2, h3, h4], axis=2)


# speedup vs baseline: 2.2451x; 1.5463x over previous
import functools

import jax
import jax.numpy as jnp
from jax import lax
from jax.experimental import pallas as pl
from jax.experimental.pallas import tpu as pltpu
from jax.experimental.pallas import tpu_sc as plsc

N = 10000
E = 320000
F = 128
HID = 16
T = 12
TOUT = 4

NC = 2
NS = 16
CH = 128
NB = 4
NP = 10112
RPT = NP // NS
DISCARD = NP - 1

PH = 3
WH = PH * 2 * HID
CPT_A = 160
EPT_A = CPT_A * CH
EPC_A = EPT_A * NS

WB = 2 * HID
EH = E // 2
CPT_B = 80
EPT_B = CPT_B * CH
EPC_B = EPT_B * NS

WD = 16

BN = 1000


def _sc_edge_scatter(width, halves, cpt):
    mesh = plsc.VectorSubcoreMesh(
        core_axis_name="c", subcore_axis_name="s",
        num_cores=NC, num_subcores=NS)

    @functools.partial(
        pl.kernel,
        out_type=jax.ShapeDtypeStruct((NC, halves, NP, width), jnp.float32),
        mesh=mesh,
        scratch_types=[
            pltpu.VMEM((NB, CH), jnp.int32),
            pltpu.VMEM((NB, CH), jnp.int32),
            pltpu.VMEM((NB, CH, width), jnp.float32),
            pltpu.VMEM_SHARED((NP, width), jnp.float32),
            [pltpu.SemaphoreType.DMA] * NB,
        ],
        compiler_params=pltpu.CompilerParams(use_tc_tiling_on_sc=False),
    )
    def k(g, src, dst, zeros, out, src_g, dst_g, rows, acc, sems):
        c = lax.axis_index("c")
        s = lax.axis_index("s")
        r0 = s * RPT
        for h in range(halves):
            pltpu.sync_copy(zeros, acc.at[pl.ds(r0, RPT)])
            plsc.subcore_barrier()

            def grp(j, carry):
                i0 = j * NB
                pltpu.sync_copy(src.at[c, h, s, pl.ds(i0, NB)], src_g)
                pltpu.sync_copy(dst.at[c, s, pl.ds(i0, NB)], dst_g)
                descs = [
                    pltpu.async_copy(g.at[src_g.at[b]], rows.at[b],
                                     sems[b])
                    for b in range(NB)
                ]
                for b in range(NB):
                    descs[b].wait()
                    pltpu.sync_copy(rows.at[b], acc.at[dst_g.at[b]],
                                    add=True)
                return carry

            lax.fori_loop(0, cpt // NB, grp, 0)
            plsc.subcore_barrier()
            pltpu.sync_copy(acc.at[pl.ds(r0, RPT)],
                            out.at[c, h, pl.ds(r0, RPT)])

    return k


def _sc_degree():
    mesh = plsc.VectorSubcoreMesh(
        core_axis_name="c", subcore_axis_name="s",
        num_cores=NC, num_subcores=NS)

    @functools.partial(
        pl.kernel,
        out_type=jax.ShapeDtypeStruct((NC, NP, WD), jnp.float32),
        mesh=mesh,
        scratch_types=[
            pltpu.VMEM((CPT_B, CH), jnp.int32),
            pltpu.VMEM((CH, WD), jnp.float32),
            pltpu.VMEM_SHARED((NP, WD), jnp.float32),
        ],
        compiler_params=pltpu.CompilerParams(use_tc_tiling_on_sc=False),
    )
    def k(dst, zeros, ones, out, dst_v, ones_v, acc):
        c = lax.axis_index("c")
        s = lax.axis_index("s")
        r0 = s * RPT
        pltpu.sync_copy(dst.at[c, s], dst_v)
        pltpu.sync_copy(ones, ones_v)
        pltpu.sync_copy(zeros, acc.at[pl.ds(r0, RPT)])
        plsc.subcore_barrier()

        def chunk(i, carry):
            pltpu.sync_copy(ones_v, acc.at[dst_v.at[i]], add=True)
            return carry

        lax.fori_loop(0, CPT_B, chunk, 0)
        plsc.subcore_barrier()
        pltpu.sync_copy(acc.at[pl.ds(r0, RPT)], out.at[c, pl.ds(r0, RPT)])

    return k


def _full_spec(arr):
    shape = arr.shape
    return pl.BlockSpec(shape, lambda i: (0,) * len(shape))


def _acat(wz, lzw, wh, lhw):
    az = jnp.dot(wz, lzw[:HID], preferred_element_type=jnp.float32)
    ah = jnp.dot(wh, lhw[:HID], preferred_element_type=jnp.float32)
    return jnp.concatenate([az, ah], axis=1)


def _softmax12(att):
    m = jnp.max(att)
    e = jnp.exp(att - m)
    return e / jnp.sum(e)


def _gate(sg, bz, lzw, lzb, bh, lhw, lhb):
    cz = jnp.dot(bz, lzw[:HID], preferred_element_type=jnp.float32) + lzb
    ch = jnp.dot(bh, lhw[:HID], preferred_element_type=jnp.float32) + lhb
    z = jax.nn.sigmoid(sg[:, :HID] + cz)
    ht = jnp.tanh(sg[:, HID:] + ch)
    return (1.0 - z) * ht


def _tc_einsum_scale(wz, lzw, wh, lhw, xt, degp):
    def body(wz_r, lzw_r, wh_r, lhw_r, xt_r, degp_r, ga_r, dinv_r):
        acat = _acat(wz_r[...], lzw_r[...], wh_r[...], lhw_r[...])
        parts = [
            jnp.dot(xt_r[q], acat, preferred_element_type=jnp.float32)
            for q in range(T)
        ]
        deg = degp_r[0] + degp_r[1] + 1.0
        dv = lax.rsqrt(deg)
        dinv_r[...] = dv
        for g2 in range(4):
            ga_r[g2] = jnp.concatenate(parts[PH * g2:PH * (g2 + 1)],
                                       axis=1) * dv

    return pl.pallas_call(
        body,
        grid=(N // BN,),
        in_specs=[
            _full_spec(wz), _full_spec(lzw), _full_spec(wh), _full_spec(lhw),
            pl.BlockSpec((T, BN, F), lambda i: (0, i, 0)),
            pl.BlockSpec((NC, BN, 1), lambda i: (0, i, 0)),
        ],
        out_specs=[
            pl.BlockSpec((4, BN, WH), lambda i: (0, i, 0)),
            pl.BlockSpec((BN, 1), lambda i: (i, 0)),
        ],
        out_shape=[
            jax.ShapeDtypeStruct((4, N, WH), jnp.float32),
            jax.ShapeDtypeStruct((N, 1), jnp.float32),
        ],
    )(wz, lzw, wh, lhw, xt, degp)


def _tc_combine(scat, ga, dinv, att, bz, lzb, bh, lhb, lzw, lhw, wz, wh,
                outw, outb):
    def body(scat_r, ga_r, dinv_r, att_r, bz_r, lzb_r, bh_r, lhb_r,
             lzw_r, lhw_r, wz_r, wh_r, outw_r, outb_r,
             h_r, g12_r, pout_r):
        probs = _softmax12(att_r[...])
        dv = dinv_r[...]
        s = []
        for q in range(T):
            g2, pp = q // PH, q % PH
            sg = (scat_r[g2][:, pp * WB:(pp + 1) * WB]
                  + ga_r[g2][:, pp * WB:(pp + 1) * WB]) * dv
            s.append(_gate(sg, bz_r[...], lzw_r[...], lzb_r[...],
                           bh_r[...], lhw_r[...], lhb_r[...]))
        ps = []
        for t in range(TOUT):
            acc = jnp.zeros_like(s[0])
            for q in range(t, T):
                acc = acc + probs[0:1, q - t:q - t + 1] * s[q]
            ps.append(acc)
        h = jnp.dot(jax.nn.relu(ps[0]), outw_r[...],
                    preferred_element_type=jnp.float32) + outb_r[...]
        h_r[...] = h
        acat = _acat(wz_r[...], lzw_r[...], wh_r[...], lhw_r[...])
        g12_r[...] = jnp.dot(h, acat, preferred_element_type=jnp.float32) * dv
        for t in range(1, TOUT):
            pout_r[t - 1] = ps[t]

    return pl.pallas_call(
        body,
        grid=(N // BN,),
        in_specs=[
            pl.BlockSpec((2 * NC, BN, WH), lambda i: (0, i, 0)),
            pl.BlockSpec((4, BN, WH), lambda i: (0, i, 0)),
            pl.BlockSpec((BN, 1), lambda i: (i, 0)),
            _full_spec(att), _full_spec(bz), _full_spec(lzb),
            _full_spec(bh), _full_spec(lhb), _full_spec(lzw),
            _full_spec(lhw), _full_spec(wz), _full_spec(wh),
            _full_spec(outw), _full_spec(outb),
        ],
        out_specs=[
            pl.BlockSpec((BN, F), lambda i: (i, 0)),
            pl.BlockSpec((BN, WB), lambda i: (i, 0)),
            pl.BlockSpec((TOUT - 1, BN, HID), lambda i: (0, i, 0)),
        ],
        out_shape=[
            jax.ShapeDtypeStruct((N, F), jnp.float32),
            jax.ShapeDtypeStruct((N, WB), jnp.float32),
            jax.ShapeDtypeStruct((TOUT - 1, N, HID), jnp.float32),
        ],
    )(scat, ga, dinv, att, bz, lzb, bh, lhb, lzw, lhw, wz, wh, outw, outb)


def _tc_step(scatb, gt, dinv, pin, att, bz, lzb, bh, lhb, lzw, lhw, wz, wh,
             outw, outb, n_p, emit_g):
    def body(*refs):
        (scatb_r, gt_r, dinv_r, pin_r, att_r, bz_r, lzb_r, bh_r, lhb_r,
         lzw_r, lhw_r, wz_r, wh_r, outw_r, outb_r) = refs[:15]
        outs = refs[15:]
        h_r = outs[0]
        probs = _softmax12(att_r[...])
        dv = dinv_r[...]
        sg = (scatb_r[0, 0] + scatb_r[1, 0] + gt_r[...]) * dv
        s_new = _gate(sg, bz_r[...], lzw_r[...], lzb_r[...],
                      bh_r[...], lhw_r[...], lhb_r[...])
        h = jnp.dot(jax.nn.relu(pin_r[0] + probs[0:1, 11:12] * s_new),
                    outw_r[...], preferred_element_type=jnp.float32)
        h = h + outb_r[...]
        h_r[...] = h
        o = 1
        if n_p > 1:
            pout_r = outs[o]
            o += 1
            for j in range(n_p - 1):
                pout_r[j] = pin_r[j + 1] + probs[0:1, 10 - j:11 - j] * s_new
        if emit_g:
            acat = _acat(wz_r[...], lzw_r[...], wh_r[...], lhw_r[...])
            outs[o][...] = jnp.dot(
                h, acat, preferred_element_type=jnp.float32) * dv

    out_specs = [pl.BlockSpec((BN, F), lambda i: (i, 0))]
    out_shape = [jax.ShapeDtypeStruct((N, F), jnp.float32)]
    if n_p > 1:
        out_specs.append(pl.BlockSpec((n_p - 1, BN, HID), lambda i: (0, i, 0)))
        out_shape.append(jax.ShapeDtypeStruct((n_p - 1, N, HID), jnp.float32))
    if emit_g:
        out_specs.append(pl.BlockSpec((BN, WB), lambda i: (i, 0)))
        out_shape.append(jax.ShapeDtypeStruct((N, WB), jnp.float32))

    return pl.pallas_call(
        body,
        grid=(N // BN,),
        in_specs=[
            pl.BlockSpec((NC, 1, BN, WB), lambda i: (0, 0, i, 0)),
            pl.BlockSpec((BN, WB), lambda i: (i, 0)),
            pl.BlockSpec((BN, 1), lambda i: (i, 0)),
            pl.BlockSpec((n_p, BN, HID), lambda i: (0, i, 0)),
            _full_spec(att), _full_spec(bz), _full_spec(lzb),
            _full_spec(bh), _full_spec(lhb), _full_spec(lzw),
            _full_spec(lhw), _full_spec(wz), _full_spec(wh),
            _full_spec(outw), _full_spec(outb),
        ],
        out_specs=out_specs,
        out_shape=out_shape,
    )(scatb, gt, dinv, pin, att, bz, lzb, bh, lhb, lzw, lhw, wz, wh,
      outw, outb)


def _pad_flat(v, total, base, mod):
    n_pad = total - v.shape[0]
    pad = base + (jnp.arange(n_pad, dtype=jnp.int32) % mod)
    return jnp.concatenate([v, pad])


def _pad_reshape(v, total, base, mod):
    return _pad_flat(v, total, base, mod).reshape(NS, -1, CH)


def kernel(x, edge_index, Wz, bz, lzW, lzb, Wr, br, lrW, lrb, Wh, bh, lhW,
           lhb, att, outW, outb):
    x = x.astype(jnp.float32)
    src = edge_index[0].astype(jnp.int32)
    dst = edge_index[1].astype(jnp.int32)

    att2 = att.reshape(1, T)
    bz2 = bz.reshape(1, HID)
    lzb2 = lzb.reshape(1, HID)
    bh2 = bh.reshape(1, HID)
    lhb2 = lhb.reshape(1, HID)
    outb2 = outb.reshape(1, F)

    n_disc = NP - N
    src_a = jnp.stack([
        jnp.stack([_pad_reshape(src + (2 * c + h) * N, EPC_A,
                                (2 * c + h) * N, N)
                   for h in range(2)])
        for c in range(NC)
    ])
    dst_a1 = _pad_reshape(dst, EPC_A, N, n_disc)
    dst_a = jnp.stack([dst_a1, dst_a1])

    src_b = jnp.stack([
        _pad_reshape(src[:EH], EPC_B, 0, N)[None],
        _pad_reshape(src[EH:], EPC_B, 0, N)[None],
    ])
    dst_b = jnp.stack([
        _pad_reshape(dst[:EH], EPC_B, N, n_disc),
        _pad_reshape(dst[EH:], EPC_B, N, n_disc),
    ])
    dst_d = jnp.stack([
        _pad_reshape(dst[:EH], EPC_B, N, n_disc),
        _pad_reshape(dst[EH:], EPC_B, N, n_disc),
    ])

    zeros_a = jnp.zeros((RPT, WH), jnp.float32)
    zeros_b = jnp.zeros((RPT, WB), jnp.float32)
    zeros_d = jnp.zeros((RPT, WD), jnp.float32)
    ones_d = jnp.ones((CH, WD), jnp.float32)

    degp_full = _sc_degree()(dst_d, zeros_d, ones_d)
    degp = degp_full[:, :N, 0:1]

    xt = jnp.transpose(x, (2, 0, 1))
    ga, dinv = _tc_einsum_scale(Wz, lzW, Wh, lhW, xt, degp)
    gaf = ga.reshape(4 * N, WH)
    scat12 = _sc_edge_scatter(WH, 2, CPT_A)(gaf, src_a, dst_a, zeros_a)
    scat12 = scat12.reshape(2 * NC, NP, WH)[:, :N]

    h1, g12, pp3 = _tc_combine(scat12, ga, dinv, att2, bz2, lzb2, bh2, lhb2,
                               lzW, lhW, Wz, Wh, outW, outb2)

    sc_b = _sc_edge_scatter(WB, 1, CPT_B)

    sb12 = sc_b(g12, src_b, dst_b, zeros_b)[:, :, :N]
    h2, pp2, g13 = _tc_step(sb12, g12, dinv, pp3, att2, bz2, lzb2, bh2, lhb2,
                            lzW, lhW, Wz, Wh, outW, outb2, 3, True)

    sb13 = sc_b(g13, src_b, dst_b, zeros_b)[:, :, :N]
    h3, pp1, g14 = _tc_step(sb13, g13, dinv, pp2, att2, bz2, lzb2, bh2, lhb2,
                            lzW, lhW, Wz, Wh, outW, outb2, 2, True)

    sb14 = sc_b(g14, src_b, dst_b, zeros_b)[:, :, :N]
    (h4,) = _tc_step(sb14, g14, dinv, pp1, att2, bz2, lzb2, bh2, lhb2,
                     lzW, lhW, Wz, Wh, outW, outb2, 1, False)

    return jnp.stack([h1, h2, h3, h4], axis=2)


# speedup vs baseline: 2.4336x; 1.0839x over previous
import functools

import jax
import jax.numpy as jnp
from jax import lax
from jax.experimental import pallas as pl
from jax.experimental.pallas import tpu as pltpu
from jax.experimental.pallas import tpu_sc as plsc

N = 10000
E = 320000
F = 128
HID = 16
T = 12
TOUT = 4

NC = 2
NS = 16
CH = 128
NB = 4
NP = 10112
RPT = NP // NS
DISCARD = NP - 1

PH = 3
WH = PH * 2 * HID
CPT_A = 160
EPT_A = CPT_A * CH
EPC_A = EPT_A * NS

WB = 2 * HID
EH = E // 2
CPT_B = 80
EPT_B = CPT_B * CH
EPC_B = EPT_B * NS

WD = 16

BN = 1000


def _sc_edge_scatter(width, halves, cpt, nb):
    mesh = plsc.VectorSubcoreMesh(
        core_axis_name="c", subcore_axis_name="s",
        num_cores=NC, num_subcores=NS)

    @functools.partial(
        pl.kernel,
        out_type=jax.ShapeDtypeStruct((NC, halves, NP, width), jnp.float32),
        mesh=mesh,
        scratch_types=[
            pltpu.VMEM((nb, CH), jnp.int32),
            pltpu.VMEM((nb, CH), jnp.int32),
            pltpu.VMEM((nb, CH, width), jnp.float32),
            pltpu.VMEM_SHARED((NP, width), jnp.float32),
            [pltpu.SemaphoreType.DMA] * nb,
        ],
        compiler_params=pltpu.CompilerParams(use_tc_tiling_on_sc=False),
    )
    def k(g, src, dst, zeros, out, src_g, dst_g, rows, acc, sems):
        c = lax.axis_index("c")
        s = lax.axis_index("s")
        r0 = s * RPT
        for h in range(halves):
            pltpu.sync_copy(zeros, acc.at[pl.ds(r0, RPT)])
            plsc.subcore_barrier()

            def grp(j, carry):
                i0 = j * nb
                pltpu.sync_copy(src.at[c, h, s, pl.ds(i0, nb)], src_g)
                pltpu.sync_copy(dst.at[c, s, pl.ds(i0, nb)], dst_g)
                descs = [
                    pltpu.async_copy(g.at[src_g.at[b]], rows.at[b],
                                     sems[b])
                    for b in range(nb)
                ]
                for b in range(nb):
                    descs[b].wait()
                    pltpu.sync_copy(rows.at[b], acc.at[dst_g.at[b]],
                                    add=True)
                return carry

            lax.fori_loop(0, cpt // nb, grp, 0)
            plsc.subcore_barrier()
            pltpu.sync_copy(acc.at[pl.ds(r0, RPT)],
                            out.at[c, h, pl.ds(r0, RPT)])

    return k


def _sc_degree():
    mesh = plsc.VectorSubcoreMesh(
        core_axis_name="c", subcore_axis_name="s",
        num_cores=NC, num_subcores=NS)

    @functools.partial(
        pl.kernel,
        out_type=jax.ShapeDtypeStruct((NC, NP, WD), jnp.float32),
        mesh=mesh,
        scratch_types=[
            pltpu.VMEM((CPT_B, CH), jnp.int32),
            pltpu.VMEM((CH, WD), jnp.float32),
            pltpu.VMEM_SHARED((NP, WD), jnp.float32),
        ],
        compiler_params=pltpu.CompilerParams(use_tc_tiling_on_sc=False),
    )
    def k(dst, zeros, ones, out, dst_v, ones_v, acc):
        c = lax.axis_index("c")
        s = lax.axis_index("s")
        r0 = s * RPT
        pltpu.sync_copy(dst.at[c, s], dst_v)
        pltpu.sync_copy(ones, ones_v)
        pltpu.sync_copy(zeros, acc.at[pl.ds(r0, RPT)])
        plsc.subcore_barrier()

        def chunk(i, carry):
            pltpu.sync_copy(ones_v, acc.at[dst_v.at[i]], add=True)
            return carry

        lax.fori_loop(0, CPT_B, chunk, 0)
        plsc.subcore_barrier()
        pltpu.sync_copy(acc.at[pl.ds(r0, RPT)], out.at[c, pl.ds(r0, RPT)])

    return k


def _full_spec(arr):
    shape = arr.shape
    return pl.BlockSpec(shape, lambda i: (0,) * len(shape))


def _acat(wz, lzw, wh, lhw):
    az = jnp.dot(wz, lzw[:HID], preferred_element_type=jnp.float32)
    ah = jnp.dot(wh, lhw[:HID], preferred_element_type=jnp.float32)
    return jnp.concatenate([az, ah], axis=1)


def _softmax12(att):
    m = jnp.max(att)
    e = jnp.exp(att - m)
    return e / jnp.sum(e)


def _gate(sg, bz, lzw, lzb, bh, lhw, lhb):
    cz = jnp.dot(bz, lzw[:HID], preferred_element_type=jnp.float32) + lzb
    ch = jnp.dot(bh, lhw[:HID], preferred_element_type=jnp.float32) + lhb
    z = jax.nn.sigmoid(sg[:, :HID] + cz)
    ht = jnp.tanh(sg[:, HID:] + ch)
    return (1.0 - z) * ht


def _tc_einsum_scale(wz, lzw, wh, lhw, xt, degp):
    def body(wz_r, lzw_r, wh_r, lhw_r, xt_r, degp_r, ga_r, dinv_r):
        acat = _acat(wz_r[...], lzw_r[...], wh_r[...], lhw_r[...])
        parts = [
            jnp.dot(xt_r[q], acat, preferred_element_type=jnp.float32)
            for q in range(T)
        ]
        deg = degp_r[0, :, 0:1] + degp_r[1, :, 0:1] + 1.0
        dv = lax.rsqrt(deg)
        dinv_r[...] = dv
        for g2 in range(4):
            ga_r[g2] = jnp.concatenate(parts[PH * g2:PH * (g2 + 1)],
                                       axis=1) * dv

    return pl.pallas_call(
        body,
        grid=(N // BN,),
        in_specs=[
            _full_spec(wz), _full_spec(lzw), _full_spec(wh), _full_spec(lhw),
            pl.BlockSpec((T, BN, F), lambda i: (0, i, 0)),
            pl.BlockSpec((NC, BN, WD), lambda i: (0, i, 0)),
        ],
        out_specs=[
            pl.BlockSpec((4, BN, WH), lambda i: (0, i, 0)),
            pl.BlockSpec((BN, 1), lambda i: (i, 0)),
        ],
        out_shape=[
            jax.ShapeDtypeStruct((4, N, WH), jnp.float32),
            jax.ShapeDtypeStruct((N, 1), jnp.float32),
        ],
    )(wz, lzw, wh, lhw, xt, degp)


def _tc_combine(scat, ga, dinv, att, bz, lzb, bh, lhb, lzw, lhw, wz, wh,
                outw, outb):
    def body(scat_r, ga_r, dinv_r, att_r, bz_r, lzb_r, bh_r, lhb_r,
             lzw_r, lhw_r, wz_r, wh_r, outw_r, outb_r,
             h_r, g12_r, pout_r):
        probs = _softmax12(att_r[...])
        dv = dinv_r[...]
        s = []
        for q in range(T):
            g2, pp = q // PH, q % PH
            sg = (scat_r[g2][:, pp * WB:(pp + 1) * WB]
                  + ga_r[g2][:, pp * WB:(pp + 1) * WB]) * dv
            s.append(_gate(sg, bz_r[...], lzw_r[...], lzb_r[...],
                           bh_r[...], lhw_r[...], lhb_r[...]))
        ps = []
        for t in range(TOUT):
            acc = jnp.zeros_like(s[0])
            for q in range(t, T):
                acc = acc + probs[0:1, q - t:q - t + 1] * s[q]
            ps.append(acc)
        h = jnp.dot(jax.nn.relu(ps[0]), outw_r[...],
                    preferred_element_type=jnp.float32) + outb_r[...]
        h_r[...] = h
        acat = _acat(wz_r[...], lzw_r[...], wh_r[...], lhw_r[...])
        g12_r[...] = jnp.dot(h, acat, preferred_element_type=jnp.float32) * dv
        for t in range(1, TOUT):
            pout_r[t - 1] = ps[t]

    return pl.pallas_call(
        body,
        grid=(N // BN,),
        in_specs=[
            pl.BlockSpec((2 * NC, BN, WH), lambda i: (0, i, 0)),
            pl.BlockSpec((4, BN, WH), lambda i: (0, i, 0)),
            pl.BlockSpec((BN, 1), lambda i: (i, 0)),
            _full_spec(att), _full_spec(bz), _full_spec(lzb),
            _full_spec(bh), _full_spec(lhb), _full_spec(lzw),
            _full_spec(lhw), _full_spec(wz), _full_spec(wh),
            _full_spec(outw), _full_spec(outb),
        ],
        out_specs=[
            pl.BlockSpec((BN, F), lambda i: (i, 0)),
            pl.BlockSpec((BN, WB), lambda i: (i, 0)),
            pl.BlockSpec((TOUT - 1, BN, HID), lambda i: (0, i, 0)),
        ],
        out_shape=[
            jax.ShapeDtypeStruct((N, F), jnp.float32),
            jax.ShapeDtypeStruct((N, WB), jnp.float32),
            jax.ShapeDtypeStruct((TOUT - 1, N, HID), jnp.float32),
        ],
    )(scat, ga, dinv, att, bz, lzb, bh, lhb, lzw, lhw, wz, wh, outw, outb)


def _tc_step(scatb, gt, dinv, pin, att, bz, lzb, bh, lhb, lzw, lhw, wz, wh,
             outw, outb, n_p, emit_g):
    def body(*refs):
        (scatb_r, gt_r, dinv_r, pin_r, att_r, bz_r, lzb_r, bh_r, lhb_r,
         lzw_r, lhw_r, wz_r, wh_r, outw_r, outb_r) = refs[:15]
        outs = refs[15:]
        h_r = outs[0]
        probs = _softmax12(att_r[...])
        dv = dinv_r[...]
        sg = (scatb_r[0, 0] + scatb_r[1, 0] + gt_r[...]) * dv
        s_new = _gate(sg, bz_r[...], lzw_r[...], lzb_r[...],
                      bh_r[...], lhw_r[...], lhb_r[...])
        h = jnp.dot(jax.nn.relu(pin_r[0] + probs[0:1, 11:12] * s_new),
                    outw_r[...], preferred_element_type=jnp.float32)
        h = h + outb_r[...]
        h_r[...] = h
        o = 1
        if n_p > 1:
            pout_r = outs[o]
            o += 1
            for j in range(n_p - 1):
                pout_r[j] = pin_r[j + 1] + probs[0:1, 10 - j:11 - j] * s_new
        if emit_g:
            acat = _acat(wz_r[...], lzw_r[...], wh_r[...], lhw_r[...])
            outs[o][...] = jnp.dot(
                h, acat, preferred_element_type=jnp.float32) * dv

    out_specs = [pl.BlockSpec((BN, F), lambda i: (i, 0))]
    out_shape = [jax.ShapeDtypeStruct((N, F), jnp.float32)]
    if n_p > 1:
        out_specs.append(pl.BlockSpec((n_p - 1, BN, HID), lambda i: (0, i, 0)))
        out_shape.append(jax.ShapeDtypeStruct((n_p - 1, N, HID), jnp.float32))
    if emit_g:
        out_specs.append(pl.BlockSpec((BN, WB), lambda i: (i, 0)))
        out_shape.append(jax.ShapeDtypeStruct((N, WB), jnp.float32))

    return pl.pallas_call(
        body,
        grid=(N // BN,),
        in_specs=[
            pl.BlockSpec((NC, 1, BN, WB), lambda i: (0, 0, i, 0)),
            pl.BlockSpec((BN, WB), lambda i: (i, 0)),
            pl.BlockSpec((BN, 1), lambda i: (i, 0)),
            pl.BlockSpec((n_p, BN, HID), lambda i: (0, i, 0)),
            _full_spec(att), _full_spec(bz), _full_spec(lzb),
            _full_spec(bh), _full_spec(lhb), _full_spec(lzw),
            _full_spec(lhw), _full_spec(wz), _full_spec(wh),
            _full_spec(outw), _full_spec(outb),
        ],
        out_specs=out_specs,
        out_shape=out_shape,
    )(scatb, gt, dinv, pin, att, bz, lzb, bh, lhb, lzw, lhw, wz, wh,
      outw, outb)


def _pad_flat(v, total, base, mod):
    n_pad = total - v.shape[0]
    pad = base + (jnp.arange(n_pad, dtype=jnp.int32) % mod)
    return jnp.concatenate([v, pad])


def _pad_reshape(v, total, base, mod):
    return _pad_flat(v, total, base, mod).reshape(NS, -1, CH)


def kernel(x, edge_index, Wz, bz, lzW, lzb, Wr, br, lrW, lrb, Wh, bh, lhW,
           lhb, att, outW, outb):
    x = x.astype(jnp.float32)
    src = edge_index[0].astype(jnp.int32)
    dst = edge_index[1].astype(jnp.int32)

    att2 = att.reshape(1, T)
    bz2 = bz.reshape(1, HID)
    lzb2 = lzb.reshape(1, HID)
    bh2 = bh.reshape(1, HID)
    lhb2 = lhb.reshape(1, HID)
    outb2 = outb.reshape(1, F)

    n_disc = NP - N
    src_a = jnp.stack([
        jnp.stack([_pad_reshape(src + (2 * c + h) * N, EPC_A,
                                (2 * c + h) * N, N)
                   for h in range(2)])
        for c in range(NC)
    ])
    dst_a1 = _pad_reshape(dst, EPC_A, N, n_disc)
    dst_a = jnp.stack([dst_a1, dst_a1])

    src_b = jnp.stack([
        _pad_reshape(src[:EH], EPC_B, 0, N)[None],
        _pad_reshape(src[EH:], EPC_B, 0, N)[None],
    ])
    dst_b = jnp.stack([
        _pad_reshape(dst[:EH], EPC_B, N, n_disc),
        _pad_reshape(dst[EH:], EPC_B, N, n_disc),
    ])
    dst_d = jnp.stack([
        _pad_reshape(dst[:EH], EPC_B, N, n_disc),
        _pad_reshape(dst[EH:], EPC_B, N, n_disc),
    ])

    zeros_a = jnp.zeros((RPT, WH), jnp.float32)
    zeros_b = jnp.zeros((RPT, WB), jnp.float32)
    zeros_d = jnp.zeros((RPT, WD), jnp.float32)
    ones_d = jnp.ones((CH, WD), jnp.float32)

    degp = _sc_degree()(dst_d, zeros_d, ones_d)

    xt = jnp.transpose(x, (2, 0, 1))
    ga, dinv = _tc_einsum_scale(Wz, lzW, Wh, lhW, xt, degp)
    gaf = ga.reshape(4 * N, WH)
    scat12 = _sc_edge_scatter(WH, 2, CPT_A, 4)(gaf, src_a, dst_a, zeros_a)
    scat12 = scat12.reshape(2 * NC, NP, WH)

    h1, g12, pp3 = _tc_combine(scat12, ga, dinv, att2, bz2, lzb2, bh2, lhb2,
                               lzW, lhW, Wz, Wh, outW, outb2)

    sc_b = _sc_edge_scatter(WB, 1, CPT_B, 8)

    sb12 = sc_b(g12, src_b, dst_b, zeros_b)
    h2, pp2, g13 = _tc_step(sb12, g12, dinv, pp3, att2, bz2, lzb2, bh2, lhb2,
                            lzW, lhW, Wz, Wh, outW, outb2, 3, True)

    sb13 = sc_b(g13, src_b, dst_b, zeros_b)
    h3, pp1, g14 = _tc_step(sb13, g13, dinv, pp2, att2, bz2, lzb2, bh2, lhb2,
                            lzW, lhW, Wz, Wh, outW, outb2, 2, True)

    sb14 = sc_b(g14, src_b, dst_b, zeros_b)
    (h4,) = _tc_step(sb14, g14, dinv, pp1, att2, bz2, lzb2, bh2, lhb2,
                     lzW, lhW, Wz, Wh, outW, outb2, 1, False)

    return jnp.stack([h1, h2, h3, h4], axis=2)


# speedup vs baseline: 2.4341x; 1.0002x over previous
import functools

import jax
import jax.numpy as jnp
from jax import lax
from jax.experimental import pallas as pl
from jax.experimental.pallas import tpu as pltpu
from jax.experimental.pallas import tpu_sc as plsc

N = 10000
E = 320000
F = 128
HID = 16
T = 12
TOUT = 4

NC = 2
NS = 16
CH = 128
NP = 10112
RPT = NP // NS

PH = 3
WH = PH * 2 * HID
CPT_A = 160
EPT_A = CPT_A * CH
EPC_A = EPT_A * NS

WB = 2 * HID
EH = E // 2
CPT_B = 80
EPT_B = CPT_B * CH
EPC_B = EPT_B * NS

WD = 16

BN = 1000


def _sc_edge_scatter(width, halves, cpt, nb):
    mesh = plsc.VectorSubcoreMesh(
        core_axis_name="c", subcore_axis_name="s",
        num_cores=NC, num_subcores=NS)

    @functools.partial(
        pl.kernel,
        out_type=jax.ShapeDtypeStruct((NC, halves, NP, width), jnp.float32),
        mesh=mesh,
        scratch_types=[
            pltpu.VMEM((nb, CH), jnp.int32),
            pltpu.VMEM((nb, CH), jnp.int32),
            pltpu.VMEM((nb, CH, width), jnp.float32),
            pltpu.VMEM_SHARED((NP, width), jnp.float32),
            [pltpu.SemaphoreType.DMA] * nb,
        ],
        compiler_params=pltpu.CompilerParams(use_tc_tiling_on_sc=False),
    )
    def k(g, src, dst, zeros, out, src_g, dst_g, rows, acc, sems):
        c = lax.axis_index("c")
        s = lax.axis_index("s")
        r0 = s * RPT
        for h in range(halves):
            pltpu.sync_copy(zeros, acc.at[pl.ds(r0, RPT)])
            plsc.subcore_barrier()

            def grp(j, carry):
                i0 = j * nb
                pltpu.sync_copy(src.at[c, h, s, pl.ds(i0, nb)], src_g)
                pltpu.sync_copy(dst.at[c, s, pl.ds(i0, nb)], dst_g)
                descs = [
                    pltpu.async_copy(g.at[src_g.at[b]], rows.at[b],
                                     sems[b])
                    for b in range(nb)
                ]
                for b in range(nb):
                    descs[b].wait()
                    pltpu.sync_copy(rows.at[b], acc.at[dst_g.at[b]],
                                    add=True)
                return carry

            lax.fori_loop(0, cpt // nb, grp, 0)
            plsc.subcore_barrier()
            pltpu.sync_copy(acc.at[pl.ds(r0, RPT)],
                            out.at[c, h, pl.ds(r0, RPT)])

    return k


def _sc_degree():
    mesh = plsc.VectorSubcoreMesh(
        core_axis_name="c", subcore_axis_name="s",
        num_cores=NC, num_subcores=NS)

    @functools.partial(
        pl.kernel,
        out_type=jax.ShapeDtypeStruct((NC, NP, WD), jnp.float32),
        mesh=mesh,
        scratch_types=[
            pltpu.VMEM((CPT_B, CH), jnp.int32),
            pltpu.VMEM((CH, WD), jnp.float32),
            pltpu.VMEM_SHARED((NP, WD), jnp.float32),
        ],
        compiler_params=pltpu.CompilerParams(use_tc_tiling_on_sc=False),
    )
    def k(dst, zeros, ones, out, dst_v, ones_v, acc):
        c = lax.axis_index("c")
        s = lax.axis_index("s")
        r0 = s * RPT
        pltpu.sync_copy(dst.at[c, s], dst_v)
        pltpu.sync_copy(ones, ones_v)
        pltpu.sync_copy(zeros, acc.at[pl.ds(r0, RPT)])
        plsc.subcore_barrier()

        def chunk(i, carry):
            pltpu.sync_copy(ones_v, acc.at[dst_v.at[i]], add=True)
            return carry

        lax.fori_loop(0, CPT_B, chunk, 0)
        plsc.subcore_barrier()
        pltpu.sync_copy(acc.at[pl.ds(r0, RPT)], out.at[c, pl.ds(r0, RPT)])

    return k


def _full_spec(arr):
    shape = arr.shape
    return pl.BlockSpec(shape, lambda i: (0,) * len(shape))


def _acat(wz, lzw, wh, lhw):
    az = jnp.dot(wz, lzw[:HID], preferred_element_type=jnp.float32)
    ah = jnp.dot(wh, lhw[:HID], preferred_element_type=jnp.float32)
    return jnp.concatenate([az, ah], axis=1)


def _softmax12(att):
    m = jnp.max(att)
    e = jnp.exp(att - m)
    return e / jnp.sum(e)


def _gate(sg, bz, lzw, lzb, bh, lhw, lhb):
    cz = jnp.dot(bz, lzw[:HID], preferred_element_type=jnp.float32) + lzb
    ch = jnp.dot(bh, lhw[:HID], preferred_element_type=jnp.float32) + lhb
    z = jax.nn.sigmoid(sg[:, :HID] + cz)
    ht = jnp.tanh(sg[:, HID:] + ch)
    return (1.0 - z) * ht


def _tc_einsum_scale(wz, lzw, wh, lhw, xt, degp):
    def body(wz_r, lzw_r, wh_r, lhw_r, xt_r, degp_r, ga_r, dinv_r):
        acat = _acat(wz_r[...], lzw_r[...], wh_r[...], lhw_r[...])
        parts = [
            jnp.dot(xt_r[q], acat, preferred_element_type=jnp.float32)
            for q in range(T)
        ]
        deg = degp_r[0, :, 0:1] + degp_r[1, :, 0:1] + 1.0
        dv = lax.rsqrt(deg)
        dinv_r[...] = dv
        for g2 in range(4):
            ga_r[g2] = jnp.concatenate(parts[PH * g2:PH * (g2 + 1)],
                                       axis=1) * dv

    return pl.pallas_call(
        body,
        grid=(N // BN,),
        in_specs=[
            _full_spec(wz), _full_spec(lzw), _full_spec(wh), _full_spec(lhw),
            pl.BlockSpec((T, BN, F), lambda i: (0, i, 0)),
            pl.BlockSpec((NC, BN, WD), lambda i: (0, i, 0)),
        ],
        out_specs=[
            pl.BlockSpec((4, BN, WH), lambda i: (0, i, 0)),
            pl.BlockSpec((BN, 1), lambda i: (i, 0)),
        ],
        out_shape=[
            jax.ShapeDtypeStruct((4, N, WH), jnp.float32),
            jax.ShapeDtypeStruct((N, 1), jnp.float32),
        ],
    )(wz, lzw, wh, lhw, xt, degp)


def _tc_combine(scat, ga, dinv, att, bz, lzb, bh, lhb, lzw, lhw, wz, wh,
                outw, outb):
    def body(scat_r, ga_r, dinv_r, att_r, bz_r, lzb_r, bh_r, lhb_r,
             lzw_r, lhw_r, wz_r, wh_r, outw_r, outb_r,
             h_r, g12_r, pout_r):
        probs = _softmax12(att_r[...])
        dv = dinv_r[...]
        s = []
        for q in range(T):
            g2, pp = q // PH, q % PH
            sg = (scat_r[g2][:, pp * WB:(pp + 1) * WB]
                  + ga_r[g2][:, pp * WB:(pp + 1) * WB]) * dv
            s.append(_gate(sg, bz_r[...], lzw_r[...], lzb_r[...],
                           bh_r[...], lhw_r[...], lhb_r[...]))
        ps = []
        for t in range(TOUT):
            acc = jnp.zeros_like(s[0])
            for q in range(t, T):
                acc = acc + probs[0:1, q - t:q - t + 1] * s[q]
            ps.append(acc)
        h = jnp.dot(jax.nn.relu(ps[0]), outw_r[...],
                    preferred_element_type=jnp.float32) + outb_r[...]
        h_r[...] = h
        acat = _acat(wz_r[...], lzw_r[...], wh_r[...], lhw_r[...])
        g12_r[...] = jnp.dot(h, acat, preferred_element_type=jnp.float32) * dv
        for t in range(1, TOUT):
            pout_r[t - 1] = ps[t]

    return pl.pallas_call(
        body,
        grid=(N // BN,),
        in_specs=[
            pl.BlockSpec((2 * NC, BN, WH), lambda i: (0, i, 0)),
            pl.BlockSpec((4, BN, WH), lambda i: (0, i, 0)),
            pl.BlockSpec((BN, 1), lambda i: (i, 0)),
            _full_spec(att), _full_spec(bz), _full_spec(lzb),
            _full_spec(bh), _full_spec(lhb), _full_spec(lzw),
            _full_spec(lhw), _full_spec(wz), _full_spec(wh),
            _full_spec(outw), _full_spec(outb),
        ],
        out_specs=[
            pl.BlockSpec((BN, F), lambda i: (i, 0)),
            pl.BlockSpec((BN, WB), lambda i: (i, 0)),
            pl.BlockSpec((TOUT - 1, BN, HID), lambda i: (0, i, 0)),
        ],
        out_shape=[
            jax.ShapeDtypeStruct((N, F), jnp.float32),
            jax.ShapeDtypeStruct((N, WB), jnp.float32),
            jax.ShapeDtypeStruct((TOUT - 1, N, HID), jnp.float32),
        ],
    )(scat, ga, dinv, att, bz, lzb, bh, lhb, lzw, lhw, wz, wh, outw, outb)


def _tc_step(scatb, gt, dinv, pin, att, bz, lzb, bh, lhb, lzw, lhw, wz, wh,
             outw, outb, n_p, emit_g):
    def body(*refs):
        (scatb_r, gt_r, dinv_r, pin_r, att_r, bz_r, lzb_r, bh_r, lhb_r,
         lzw_r, lhw_r, wz_r, wh_r, outw_r, outb_r) = refs[:15]
        outs = refs[15:]
        h_r = outs[0]
        probs = _softmax12(att_r[...])
        dv = dinv_r[...]
        sg = (scatb_r[0, 0] + scatb_r[1, 0] + gt_r[...]) * dv
        s_new = _gate(sg, bz_r[...], lzw_r[...], lzb_r[...],
                      bh_r[...], lhw_r[...], lhb_r[...])
        h = jnp.dot(jax.nn.relu(pin_r[0] + probs[0:1, 11:12] * s_new),
                    outw_r[...], preferred_element_type=jnp.float32)
        h = h + outb_r[...]
        h_r[...] = h
        o = 1
        if n_p > 1:
            pout_r = outs[o]
            o += 1
            for j in range(n_p - 1):
                pout_r[j] = pin_r[j + 1] + probs[0:1, 10 - j:11 - j] * s_new
        if emit_g:
            acat = _acat(wz_r[...], lzw_r[...], wh_r[...], lhw_r[...])
            outs[o][...] = jnp.dot(
                h, acat, preferred_element_type=jnp.float32) * dv

    out_specs = [pl.BlockSpec((BN, F), lambda i: (i, 0))]
    out_shape = [jax.ShapeDtypeStruct((N, F), jnp.float32)]
    if n_p > 1:
        out_specs.append(pl.BlockSpec((n_p - 1, BN, HID), lambda i: (0, i, 0)))
        out_shape.append(jax.ShapeDtypeStruct((n_p - 1, N, HID), jnp.float32))
    if emit_g:
        out_specs.append(pl.BlockSpec((BN, WB), lambda i: (i, 0)))
        out_shape.append(jax.ShapeDtypeStruct((N, WB), jnp.float32))

    return pl.pallas_call(
        body,
        grid=(N // BN,),
        in_specs=[
            pl.BlockSpec((NC, 1, BN, WB), lambda i: (0, 0, i, 0)),
            pl.BlockSpec((BN, WB), lambda i: (i, 0)),
            pl.BlockSpec((BN, 1), lambda i: (i, 0)),
            pl.BlockSpec((n_p, BN, HID), lambda i: (0, i, 0)),
            _full_spec(att), _full_spec(bz), _full_spec(lzb),
            _full_spec(bh), _full_spec(lhb), _full_spec(lzw),
            _full_spec(lhw), _full_spec(wz), _full_spec(wh),
            _full_spec(outw), _full_spec(outb),
        ],
        out_specs=out_specs,
        out_shape=out_shape,
    )(scatb, gt, dinv, pin, att, bz, lzb, bh, lhb, lzw, lhw, wz, wh,
      outw, outb)


def _pad_flat(v, total, base, mod):
    n_pad = total - v.shape[0]
    pad = base + (jnp.arange(n_pad, dtype=jnp.int32) % mod)
    return jnp.concatenate([v, pad])


def _pad_reshape(v, total, base, mod):
    return _pad_flat(v, total, base, mod).reshape(NS, -1, CH)


def kernel(x, edge_index, Wz, bz, lzW, lzb, Wr, br, lrW, lrb, Wh, bh, lhW,
           lhb, att, outW, outb):
    x = x.astype(jnp.float32)
    src = edge_index[0].astype(jnp.int32)
    dst = edge_index[1].astype(jnp.int32)

    att2 = att.reshape(1, T)
    bz2 = bz.reshape(1, HID)
    lzb2 = lzb.reshape(1, HID)
    bh2 = bh.reshape(1, HID)
    lhb2 = lhb.reshape(1, HID)
    outb2 = outb.reshape(1, F)

    n_disc = NP - N
    src_a = jnp.stack([
        jnp.stack([_pad_reshape(src + (2 * c + h) * N, EPC_A,
                                (2 * c + h) * N, N)
                   for h in range(2)])
        for c in range(NC)
    ])
    dst_a1 = _pad_reshape(dst, EPC_A, N, n_disc)
    dst_a = jnp.stack([dst_a1, dst_a1])

    src_b = jnp.stack([
        _pad_reshape(src[:EH], EPC_B, 0, N)[None],
        _pad_reshape(src[EH:], EPC_B, 0, N)[None],
    ])
    dst_b = jnp.stack([
        _pad_reshape(dst[:EH], EPC_B, N, n_disc),
        _pad_reshape(dst[EH:], EPC_B, N, n_disc),
    ])
    dst_d = jnp.stack([
        _pad_reshape(dst[:EH], EPC_B, N, n_disc),
        _pad_reshape(dst[EH:], EPC_B, N, n_disc),
    ])

    zeros_a = jnp.zeros((RPT, WH), jnp.float32)
    zeros_b = jnp.zeros((RPT, WB), jnp.float32)
    zeros_d = jnp.zeros((RPT, WD), jnp.float32)
    ones_d = jnp.ones((CH, WD), jnp.float32)

    degp = _sc_degree()(dst_d, zeros_d, ones_d)

    xt = jnp.transpose(x, (2, 0, 1))
    ga, dinv = _tc_einsum_scale(Wz, lzW, Wh, lhW, xt, degp)
    gaf = ga.reshape(4 * N, WH)
    scat12 = _sc_edge_scatter(WH, 2, CPT_A, 4)(gaf, src_a, dst_a, zeros_a)
    scat12 = scat12.reshape(2 * NC, NP, WH)

    h1, g12, pp3 = _tc_combine(scat12, ga, dinv, att2, bz2, lzb2, bh2, lhb2,
                               lzW, lhW, Wz, Wh, outW, outb2)

    sc_b = _sc_edge_scatter(WB, 1, CPT_B, 8)

    sb12 = sc_b(g12, src_b, dst_b, zeros_b)
    h2, pp2, g13 = _tc_step(sb12, g12, dinv, pp3, att2, bz2, lzb2, bh2, lhb2,
                            lzW, lhW, Wz, Wh, outW, outb2, 3, True)

    sb13 = sc_b(g13, src_b, dst_b, zeros_b)
    h3, pp1, g14 = _tc_step(sb13, g13, dinv, pp2, att2, bz2, lzb2, bh2, lhb2,
                            lzW, lhW, Wz, Wh, outW, outb2, 2, True)

    sb14 = sc_b(g14, src_b, dst_b, zeros_b)
    (h4,) = _tc_step(sb14, g14, dinv, pp1, att2, bz2, lzb2, bh2, lhb2,
                     lzW, lhW, Wz, Wh, outW, outb2, 1, False)

    return jnp.stack([h1, h2, h3, h4], axis=2)
